# baseline (speedup 1.0000x reference)
import functools

import jax
import jax.numpy as jnp
from jax import lax
from jax.experimental import pallas as pl
from jax.experimental.pallas import tpu as pltpu
from jax.experimental.pallas import tpu_sc as plsc

_FIELD_SIZE = 100000
_F = 26
_D = 16
_B = 16384
_NC = 2
_NS = 16
_NW = _NC * _NS
_BPW = _B // _NW
_C = 128
_NK = _BPW // _C
_ROWS = _C * _F
_NDMA = _ROWS // _C


def _fm_kernel(idx_hbm, t1_hbm, t2_hbm, bias_hbm, out_hbm,
               idx_v, e2_v, e1_v, zbuf, bias_v, sem1, sem2):
    wid = lax.axis_index("s") * _NC + lax.axis_index("c")

    pltpu.sync_copy(bias_hbm, bias_v)
    lane = lax.iota(jnp.int32, _D)

    for k in range(_NK):
        pltpu.sync_copy(idx_hbm.at[wid, k], idx_v)

        copies = []
        for j in range(_NDMA):
            copies.append(pltpu.async_copy(
                t2_hbm.at[idx_v.at[j]],
                e2_v.at[pl.ds(j * _C, _C)], sem2))
            copies.append(pltpu.async_copy(
                t1_hbm.at[idx_v.at[j]],
                e1_v.at[pl.ds(j * _C, _C)], sem1))
        for cp in copies:
            cp.wait()

        bvec = bias_v[...]

        def group_body(g, carry):
            acc = jnp.zeros((_D,), jnp.float32)
            for j in range(_D):
                base = (g * _D + j) * _F
                s = jnp.zeros((_D,), jnp.float32)
                sq = jnp.zeros((_D,), jnp.float32)
                for f in range(_F):
                    r = e2_v[base + f, :]
                    s = s + r
                    sq = sq + r * r
                fm = 0.5 * jnp.sum(s * s - sq)
                va = e1_v[pl.ds(base, _D)]
                vb = e1_v[pl.ds(base + _D, _D)]
                vb = jnp.where(lane < (_F - _D), vb, 0.0)
                z = fm + jnp.sum(va + vb)
                acc = jnp.where(lane == j, z, acc)
            zv = acc + bvec
            zbuf[pl.ds(g * _D, _D)] = 1.0 / (1.0 + jnp.exp(-zv))
            return carry

        lax.fori_loop(0, _C // _D, group_body, 0)

        pltpu.sync_copy(zbuf, out_hbm.at[pl.ds(wid * _BPW + k * _C, _C)])


@jax.jit
def kernel(x, table1, table2, bias):
    offsets = jnp.arange(_F, dtype=x.dtype) * _FIELD_SIZE
    flat = (x + offsets[None, :]).astype(jnp.int32)
    idx = flat.reshape(_NW, _NK, _NDMA, _C)

    mesh = plsc.VectorSubcoreMesh(core_axis_name="c", subcore_axis_name="s")
    run = functools.partial(
        pl.kernel,
        mesh=mesh,
        compiler_params=pltpu.CompilerParams(needs_layout_passes=False,
                                             use_tc_tiling_on_sc=False),
        out_type=jax.ShapeDtypeStruct((_B,), jnp.float32),
        scratch_types=[
            pltpu.VMEM((_NDMA, _C), jnp.int32),
            pltpu.VMEM((_ROWS, _D), jnp.float32),
            pltpu.VMEM((_ROWS + _D,), jnp.float32),
            pltpu.VMEM((_C,), jnp.float32),
            pltpu.VMEM((_D,), jnp.float32),
            pltpu.SemaphoreType.DMA,
            pltpu.SemaphoreType.DMA,
        ],
    )(_fm_kernel)
    bias16 = jnp.broadcast_to(bias.astype(jnp.float32), (_D,))
    out = run(idx, table1[:, 0], table2, bias16)
    return out[:, None]
